# baseline (speedup 1.0000x reference)
import functools

import jax
import jax.numpy as jnp
from jax import lax
from jax.experimental import pallas as pl
from jax.experimental.pallas import tpu as pltpu
from jax.experimental.pallas import tpu_sc as plsc

N = 2048
E = 8
T = 256
NT = N // T + E
N_PAD = NT * T
D = 256
RB = 256


def _relu(x):
    return jnp.maximum(x, 0.0)


def _routing_body(oh_ref, pos_ref, aux_ref):
    ri = lax.broadcasted_iota(jnp.int32, (RB, RB), 0)
    ci = lax.broadcasted_iota(jnp.int32, (RB, RB), 1)
    slt = jnp.where(ri > ci, 1.0, 0.0)
    carry = jnp.zeros((1, 16), jnp.float32)
    ranks = []
    ohs = []
    for blk in range(N // RB):
        o = oh_ref[blk * RB:(blk + 1) * RB, :]
        csum_excl = jnp.dot(slt, o, preferred_element_type=jnp.float32) + carry
        carry = carry + jnp.sum(o, axis=0, keepdims=True)
        ranks.append(csum_excl * o)
        ohs.append(o)
    counts = carry
    padded = jnp.floor((counts + (T - 1)) * (1.0 / T)) * T
    gi = lax.broadcasted_iota(jnp.int32, (16, 16), 0)
    gj = lax.broadcasted_iota(jnp.int32, (16, 16), 1)
    same_group = (gi < 8) == (gj < 8)
    slt16 = jnp.where((gi < gj) & same_group, 1.0, 0.0)
    off = jnp.dot(padded, slt16, preferred_element_type=jnp.float32)
    for blk in range(N // RB):
        o = ohs[blk]
        slot = ranks[blk] + o * off
        pos0 = jnp.sum(slot[:, 0:8], axis=1, keepdims=True)
        pos1 = jnp.sum(slot[:, 8:16], axis=1, keepdims=True)
        pos_ref[blk * RB:(blk + 1) * RB, :] = jnp.concatenate(
            [pos0, pos1], axis=1).astype(jnp.int32)
    aux = jnp.concatenate([off, padded, jnp.zeros((6, 16), jnp.float32)], axis=0)
    aux_ref[...] = aux.astype(jnp.int32)


def _routing(oh01):
    return pl.pallas_call(
        _routing_body,
        in_specs=[pl.BlockSpec((N, 16), lambda: (0, 0))],
        out_specs=[pl.BlockSpec((N, 2), lambda: (0, 0)),
                   pl.BlockSpec((8, 16), lambda: (0, 0))],
        out_shape=[jax.ShapeDtypeStruct((N, 2), jnp.int32),
                   jax.ShapeDtypeStruct((8, 16), jnp.int32)],
    )(oh01)


def _mlp0_body(texp_ref, tval_ref, x_ref, wa_ref, ba_ref, wb_ref, bb_ref,
               wc_ref, bc_ref, out_ref):
    i = pl.program_id(0)

    @pl.when(tval_ref[i] > 0)
    def _():
        te = texp_ref[i]
        h = _relu(jnp.dot(x_ref[...], wa_ref[te], preferred_element_type=jnp.float32) + ba_ref[te])
        h = _relu(jnp.dot(h, wb_ref[te], preferred_element_type=jnp.float32) + bb_ref[te])
        out_ref[...] = _relu(jnp.dot(h, wc_ref[te], preferred_element_type=jnp.float32) + bc_ref[te])


def _mlp1_body(texp_ref, tval_ref, x_ref, prev_ref, wp_ref, bp_ref, wi_ref,
               bi_ref, wo_ref, bo_ref, out_ref):
    i = pl.program_id(0)

    @pl.when(tval_ref[i] > 0)
    def _():
        te = texp_ref[i]
        p = _relu(jnp.dot(x_ref[...], wp_ref[te], preferred_element_type=jnp.float32) + bp_ref[te])
        h1 = jnp.dot(prev_ref[...], wi_ref[te, 0:256, :], preferred_element_type=jnp.float32)
        h1 += jnp.dot(p, wi_ref[te, 256:768, :], preferred_element_type=jnp.float32)
        h1 = _relu(h1 + bi_ref[te])
        out_ref[...] = jnp.dot(h1, wo_ref[te], preferred_element_type=jnp.float32) + bo_ref[te]


def _full3(d0, d1, d2):
    return pl.BlockSpec((d0, d1, d2), lambda i, te, tv: (0, 0, 0))


def _tile_mlp0(texp, tval, x_s, W0a, b0a3, W0b, b0b3, W0c, b0c3):
    spec = pltpu.PrefetchScalarGridSpec(
        num_scalar_prefetch=2,
        grid=(NT,),
        in_specs=[
            pl.BlockSpec((T, D), lambda i, te, tv: (i, 0)),
            _full3(E, 256, 512), _full3(E, 1, 512),
            _full3(E, 512, 512), _full3(E, 1, 512),
            _full3(E, 512, 256), _full3(E, 1, 256),
        ],
        out_specs=pl.BlockSpec((T, D), lambda i, te, tv: (i, 0)),
    )
    return pl.pallas_call(
        _mlp0_body,
        grid_spec=spec,
        out_shape=jax.ShapeDtypeStruct((N_PAD, D), jnp.float32),
    )(texp, tval, x_s, W0a, b0a3, W0b, b0b3, W0c, b0c3)


def _tile_mlp1(texp, tval, x_s, prev_s, W1pre, b1pre3, W1int, b1int3, W1out, b1out3):
    spec = pltpu.PrefetchScalarGridSpec(
        num_scalar_prefetch=2,
        grid=(NT,),
        in_specs=[
            pl.BlockSpec((T, D), lambda i, te, tv: (i, 0)),
            pl.BlockSpec((T, D), lambda i, te, tv: (i, 0)),
            _full3(E, 256, 512), _full3(E, 1, 512),
            _full3(E, 768, 512), _full3(E, 1, 512),
            _full3(E, 512, 256), _full3(E, 1, 256),
        ],
        out_specs=pl.BlockSpec((T, D), lambda i, te, tv: (i, 0)),
    )
    return pl.pallas_call(
        _mlp1_body,
        grid_spec=spec,
        out_shape=jax.ShapeDtypeStruct((N_PAD, D), jnp.float32),
    )(texp, tval, x_s, prev_s, W1pre, b1pre3, W1int, b1int3, W1out, b1out3)


def _make_sc_kernels():
    info = plsc.get_sparse_core_info()
    nc, ns = info.num_cores, info.num_subcores
    nw = nc * ns
    tok_w = N // nw
    mesh = plsc.VectorSubcoreMesh(core_axis_name="c", subcore_axis_name="s")

    def _wid():
        return lax.axis_index("s") * nc + lax.axis_index("c")

    @functools.partial(
        pl.kernel, mesh=mesh,
        out_type=[jax.ShapeDtypeStruct((N_PAD, D), jnp.float32),
                  jax.ShapeDtypeStruct((N_PAD, D), jnp.float32)],
        scratch_types=[
            pltpu.VMEM((tok_w,), jnp.int32), pltpu.VMEM((tok_w,), jnp.int32),
            pltpu.VMEM((tok_w, D), jnp.float32), pltpu.VMEM((tok_w, D), jnp.float32),
            pltpu.SemaphoreType.DMA, pltpu.SemaphoreType.DMA,
            pltpu.SemaphoreType.DMA, pltpu.SemaphoreType.DMA,
        ],
    )
    def scatter_in(x0_hbm, x1_hbm, pos0_hbm, pos1_hbm, x0s_hbm, x1s_hbm,
                   idx0_v, idx1_v, r0_v, r1_v, s0, s1, s2, s3):
        base = _wid() * tok_w
        c0 = pltpu.async_copy(x0_hbm.at[pl.ds(base, tok_w)], r0_v, s0)
        c1 = pltpu.async_copy(x1_hbm.at[pl.ds(base, tok_w)], r1_v, s1)
        c2 = pltpu.async_copy(pos0_hbm.at[pl.ds(base, tok_w)], idx0_v, s2)
        c3 = pltpu.async_copy(pos1_hbm.at[pl.ds(base, tok_w)], idx1_v, s3)
        c0.wait()
        c2.wait()
        c4 = pltpu.async_copy(r0_v, x0s_hbm.at[idx0_v], s0)
        c1.wait()
        c3.wait()
        c5 = pltpu.async_copy(r1_v, x1s_hbm.at[idx1_v], s1)
        c4.wait()
        c5.wait()

    @functools.partial(
        pl.kernel, mesh=mesh,
        out_type=jax.ShapeDtypeStruct((N_PAD, D), jnp.float32),
        scratch_types=[
            pltpu.VMEM((tok_w,), jnp.int32), pltpu.VMEM((tok_w,), jnp.int32),
            pltpu.VMEM((tok_w, D), jnp.float32),
            pltpu.SemaphoreType.DMA, pltpu.SemaphoreType.DMA,
        ],
    )
    def permute(h0s_hbm, pos0_hbm, pos1_hbm, prevs_hbm, idx0_v, idx1_v, rows_v, s0, s1):
        base = _wid() * tok_w
        c0 = pltpu.async_copy(pos0_hbm.at[pl.ds(base, tok_w)], idx0_v, s0)
        c1 = pltpu.async_copy(pos1_hbm.at[pl.ds(base, tok_w)], idx1_v, s1)
        c0.wait()
        pltpu.async_copy(h0s_hbm.at[idx0_v], rows_v, s0).wait()
        c1.wait()
        pltpu.async_copy(rows_v, prevs_hbm.at[idx1_v], s1).wait()

    @functools.partial(
        pl.kernel, mesh=mesh,
        out_type=jax.ShapeDtypeStruct((N, D), jnp.float32),
        scratch_types=[
            pltpu.VMEM((tok_w,), jnp.int32),
            pltpu.VMEM((tok_w, D), jnp.float32),
            pltpu.SemaphoreType.DMA,
        ],
    )
    def gather_out(o1s_hbm, pos1_hbm, out_hbm, idx_v, rows_v, sem):
        base = _wid() * tok_w
        pltpu.sync_copy(pos1_hbm.at[pl.ds(base, tok_w)], idx_v)
        pltpu.async_copy(o1s_hbm.at[idx_v], rows_v, sem).wait()
        pltpu.sync_copy(rows_v, out_hbm.at[pl.ds(base, tok_w)])

    return scatter_in, permute, gather_out


def kernel(input_val, W0a, b0a, W0b, b0b, W0c, b0c, W1pre, b1pre, W1int, b1int, W1out, b1out):
    x0 = input_val[:, 0:256]
    x1 = input_val[:, 256:512]
    oh01 = input_val[:, 512:528]

    pos, aux = _routing(oh01)
    pos0 = pos[:, 0]
    pos1 = pos[:, 1]
    off = aux[0]
    padded = aux[1]
    starts = jnp.arange(NT, dtype=jnp.int32)[:, None] * T
    texp0 = jnp.sum((off[None, 0:8] <= starts).astype(jnp.int32), axis=1) - 1
    texp1 = jnp.sum((off[None, 8:16] <= starts).astype(jnp.int32), axis=1) - 1
    tval0 = (starts[:, 0] < off[7] + padded[7]).astype(jnp.int32)
    tval1 = (starts[:, 0] < off[15] + padded[15]).astype(jnp.int32)

    b0a3, b0b3, b0c3 = b0a[:, None, :], b0b[:, None, :], b0c[:, None, :]
    b1pre3, b1int3, b1out3 = b1pre[:, None, :], b1int[:, None, :], b1out[:, None, :]

    scatter_in, permute, gather_out = _make_sc_kernels()

    x0_s, x1_s = scatter_in(x0, x1, pos0, pos1)
    h0_s = _tile_mlp0(texp0, tval0, x0_s, W0a, b0a3, W0b, b0b3, W0c, b0c3)
    prev_s = permute(h0_s, pos0, pos1)
    o1_s = _tile_mlp1(texp1, tval1, x1_s, prev_s, W1pre, b1pre3, W1int, b1int3, W1out, b1out3)
    return gather_out(o1_s, pos1)
